# Initial kernel scaffold; baseline (speedup 1.0000x reference)
import functools

import jax
import jax.numpy as jnp
from jax import lax
from jax.experimental import pallas as pl
from jax.experimental.pallas import tpu as pltpu
from jax.experimental.pallas import tpu_sc as plsc

VOCAB = 1000000
EMB_DIM = 64
BATCH = 16384
HIST = 50

NUM_CORES = 2
NUM_SUBCORES = 16
NW = NUM_CORES * NUM_SUBCORES

B = BATCH * HIST
BPW = B // NW
CHUNK = 512
NCHUNK = BPW // CHUNK

_mesh = plsc.VectorSubcoreMesh(
    core_axis_name="c", subcore_axis_name="s",
    num_cores=NUM_CORES, num_subcores=NUM_SUBCORES,
)


@functools.partial(
    pl.kernel,
    out_type=jax.ShapeDtypeStruct((B, EMB_DIM), jnp.float32),
    mesh=_mesh,
    scratch_types=[
        pltpu.VMEM((CHUNK,), jnp.int32),
        pltpu.VMEM((CHUNK, EMB_DIM), jnp.float32),
        pltpu.SemaphoreType.DMA,
    ],
)
def _gather_kernel(idx_hbm, table_hbm, out_hbm, idx_v, rows_v, sem):
    wid = lax.axis_index("s") * NUM_CORES + lax.axis_index("c")
    base = wid * BPW

    def body(i, carry):
        off = base + i * CHUNK
        pltpu.sync_copy(idx_hbm.at[pl.ds(off, CHUNK)], idx_v)
        pltpu.async_copy(table_hbm.at[idx_v], rows_v, sem).wait()
        pltpu.sync_copy(rows_v, out_hbm.at[pl.ds(off, CHUNK)])
        return carry

    lax.fori_loop(0, NCHUNK, body, 0)


def kernel(x, emb_weight):
    idx = x.reshape(B)
    out = _gather_kernel(idx, emb_weight)
    return out.reshape(BATCH, HIST, EMB_DIM)


# speedup vs baseline: 1.7972x; 1.7972x over previous
import functools

import jax
import jax.numpy as jnp
from jax import lax
from jax.experimental import pallas as pl
from jax.experimental.pallas import tpu as pltpu
from jax.experimental.pallas import tpu_sc as plsc

VOCAB = 1000000
EMB_DIM = 64
BATCH = 16384
HIST = 50

NUM_CORES = 2
NUM_SUBCORES = 16
NW = NUM_CORES * NUM_SUBCORES

B = BATCH * HIST
BPW = B // NW
CHUNK = 512
NCHUNK = BPW // CHUNK

_mesh = plsc.VectorSubcoreMesh(
    core_axis_name="c", subcore_axis_name="s",
    num_cores=NUM_CORES, num_subcores=NUM_SUBCORES,
)


@functools.partial(
    pl.kernel,
    out_type=jax.ShapeDtypeStruct((B, EMB_DIM), jnp.float32),
    mesh=_mesh,
    scratch_types=[
        pltpu.VMEM((CHUNK,), jnp.int32),
        pltpu.VMEM((CHUNK, EMB_DIM), jnp.float32),
        pltpu.SemaphoreType.DMA,
    ],
    compiler_params=pltpu.CompilerParams(use_tc_tiling_on_sc=False),
)
def _gather_kernel(idx_hbm, table_hbm, out_hbm, idx_v, rows_v, sem):
    wid = lax.axis_index("s") * NUM_CORES + lax.axis_index("c")
    base = wid * BPW

    def body(i, carry):
        off = base + i * CHUNK
        pltpu.sync_copy(idx_hbm.at[pl.ds(off, CHUNK)], idx_v)
        pltpu.async_copy(table_hbm.at[idx_v], rows_v, sem).wait()
        pltpu.sync_copy(rows_v, out_hbm.at[pl.ds(off, CHUNK)])
        return carry

    lax.fori_loop(0, NCHUNK, body, 0)


def kernel(x, emb_weight):
    idx = x.reshape(B)
    out = _gather_kernel(idx, emb_weight)
    return out.reshape(BATCH, HIST, EMB_DIM)


# speedup vs baseline: 2.5255x; 1.4052x over previous
import functools

import jax
import jax.numpy as jnp
from jax import lax
from jax.experimental import pallas as pl
from jax.experimental.pallas import tpu as pltpu
from jax.experimental.pallas import tpu_sc as plsc

VOCAB = 1000000
EMB_DIM = 64
BATCH = 16384
HIST = 50

NUM_CORES = 2
NUM_SUBCORES = 16
NW = NUM_CORES * NUM_SUBCORES

B = BATCH * HIST
BPW = B // NW
BATCH_PER_W = BPW // HIST

NB = 8
CHUNK = NB * HIST
NCHUNK = BPW // CHUNK

HIST_PAD = 56
ROW_PAD = 128
OUT_ROWS = BATCH * HIST_PAD

_mesh = plsc.VectorSubcoreMesh(
    core_axis_name="c", subcore_axis_name="s",
    num_cores=NUM_CORES, num_subcores=NUM_SUBCORES,
)


@functools.partial(
    pl.kernel,
    out_type=jax.ShapeDtypeStruct((OUT_ROWS, ROW_PAD), jnp.float32),
    mesh=_mesh,
    scratch_types=[
        pltpu.VMEM((BPW,), jnp.int32),
        pltpu.VMEM((CHUNK, EMB_DIM), jnp.float32),
        pltpu.VMEM((CHUNK, EMB_DIM), jnp.float32),
        pltpu.SemaphoreType.DMA,
        pltpu.SemaphoreType.DMA,
        pltpu.SemaphoreType.DMA,
        pltpu.SemaphoreType.DMA,
    ],
    compiler_params=pltpu.CompilerParams(use_tc_tiling_on_sc=False),
)
def _gather_kernel(idx_hbm, table_hbm, out_hbm, idx_all, rows0, rows1,
                   g0, g1, s0, s1):
    wid = lax.axis_index("s") * NUM_CORES + lax.axis_index("c")
    base = wid * BPW
    bb0 = wid * BATCH_PER_W

    rows = (rows0, rows1)
    gsem = (g0, g1)
    ssem = (s0, s1)

    pltpu.sync_copy(idx_hbm.at[pl.ds(base, BPW)], idx_all)

    def start_gather(i, slot):
        pltpu.async_copy(
            table_hbm.at[idx_all.at[pl.ds(i * CHUNK, CHUNK)]], rows[slot],
            gsem[slot])

    def wait_gather(i, slot):
        pltpu.make_async_copy(
            table_hbm.at[idx_all.at[pl.ds(i * CHUNK, CHUNK)]], rows[slot],
            gsem[slot]).wait()

    def store_chunk(i, slot):
        for j in range(NB):
            bb = bb0 + i * NB + j
            pltpu.async_copy(
                rows[slot].at[pl.ds(j * HIST, HIST)],
                out_hbm.at[pl.ds(bb * HIST_PAD, HIST), pl.ds(0, EMB_DIM)],
                ssem[slot])

    def drain_stores(i, slot):
        for j in range(NB):
            bb = bb0 + i * NB + j
            pltpu.make_async_copy(
                rows[slot].at[pl.ds(j * HIST, HIST)],
                out_hbm.at[pl.ds(bb * HIST_PAD, HIST), pl.ds(0, EMB_DIM)],
                ssem[slot]).wait()

    start_gather(0, 0)

    def pair(k, carry):
        i = 2 * k

        @pl.when(k >= 1)
        def _():
            drain_stores(i - 1, 1)
        start_gather(i + 1, 1)
        wait_gather(i, 0)
        store_chunk(i, 0)

        @pl.when(k < NCHUNK // 2 - 1)
        def _():
            drain_stores(i, 0)
            start_gather(i + 2, 0)
        wait_gather(i + 1, 1)
        store_chunk(i + 1, 1)
        return carry

    lax.fori_loop(0, NCHUNK // 2, pair, 0)
    drain_stores(NCHUNK - 2, 0)
    drain_stores(NCHUNK - 1, 1)


def kernel(x, emb_weight):
    idx = x.reshape(B)
    out = _gather_kernel(idx, emb_weight)
    out = out.reshape(BATCH, HIST_PAD, ROW_PAD)
    return out[:, :HIST, :EMB_DIM]


# speedup vs baseline: 2.5259x; 1.0002x over previous
import functools

import jax
import jax.numpy as jnp
from jax import lax
from jax.experimental import pallas as pl
from jax.experimental.pallas import tpu as pltpu
from jax.experimental.pallas import tpu_sc as plsc

VOCAB = 1000000
EMB_DIM = 64
BATCH = 16384
HIST = 50

NUM_CORES = 2
NUM_SUBCORES = 16
NW = NUM_CORES * NUM_SUBCORES

B = BATCH * HIST
BPW = B // NW
BATCH_PER_W = BPW // HIST

NB = 8
CHUNK = NB * HIST
NCHUNK = BPW // CHUNK

HIST_PAD = 56
ROW_PAD = 128
OUT_ROWS = BATCH * HIST_PAD

_mesh = plsc.VectorSubcoreMesh(
    core_axis_name="c", subcore_axis_name="s",
    num_cores=NUM_CORES, num_subcores=NUM_SUBCORES,
)


@functools.partial(
    pl.kernel,
    out_type=jax.ShapeDtypeStruct((BATCH, HIST_PAD, ROW_PAD), jnp.float32),
    mesh=_mesh,
    scratch_types=[
        pltpu.VMEM((BPW,), jnp.int32),
        pltpu.VMEM((CHUNK, EMB_DIM), jnp.float32),
        pltpu.VMEM((CHUNK, EMB_DIM), jnp.float32),
        pltpu.SemaphoreType.DMA,
        pltpu.SemaphoreType.DMA,
        pltpu.SemaphoreType.DMA,
        pltpu.SemaphoreType.DMA,
    ],
    compiler_params=pltpu.CompilerParams(use_tc_tiling_on_sc=False),
)
def _gather_kernel(idx_hbm, table_hbm, out_hbm, idx_all, rows0, rows1,
                   g0, g1, s0, s1):
    wid = lax.axis_index("s") * NUM_CORES + lax.axis_index("c")
    base = wid * BPW
    bb0 = wid * BATCH_PER_W

    rows = (rows0, rows1)
    gsem = (g0, g1)
    ssem = (s0, s1)

    pltpu.sync_copy(idx_hbm.at[pl.ds(base, BPW)], idx_all)

    def start_gather(i, slot):
        pltpu.async_copy(
            table_hbm.at[idx_all.at[pl.ds(i * CHUNK, CHUNK)]], rows[slot],
            gsem[slot])

    def wait_gather(i, slot):
        pltpu.make_async_copy(
            table_hbm.at[idx_all.at[pl.ds(i * CHUNK, CHUNK)]], rows[slot],
            gsem[slot]).wait()

    def store_chunk(i, slot):
        for j in range(NB):
            bb = bb0 + i * NB + j
            pltpu.async_copy(
                rows[slot].at[pl.ds(j * HIST, HIST)],
                out_hbm.at[bb, pl.ds(0, HIST), pl.ds(0, EMB_DIM)],
                ssem[slot])

    def drain_stores(i, slot):
        for j in range(NB):
            bb = bb0 + i * NB + j
            pltpu.make_async_copy(
                rows[slot].at[pl.ds(j * HIST, HIST)],
                out_hbm.at[bb, pl.ds(0, HIST), pl.ds(0, EMB_DIM)],
                ssem[slot]).wait()

    start_gather(0, 0)

    def pair(k, carry):
        i = 2 * k

        @pl.when(k >= 1)
        def _():
            drain_stores(i - 1, 1)
        start_gather(i + 1, 1)
        wait_gather(i, 0)
        store_chunk(i, 0)

        @pl.when(k < NCHUNK // 2 - 1)
        def _():
            drain_stores(i, 0)
            start_gather(i + 2, 0)
        wait_gather(i + 1, 1)
        store_chunk(i + 1, 1)
        return carry

    lax.fori_loop(0, NCHUNK // 2, pair, 0)
    drain_stores(NCHUNK - 2, 0)
    drain_stores(NCHUNK - 1, 1)


def kernel(x, emb_weight):
    idx = x.reshape(B)
    out = _gather_kernel(idx, emb_weight)
    return out[:, :HIST, :EMB_DIM]
